# baseline (speedup 1.0000x reference)
import jax
import jax.numpy as jnp
from jax.experimental import pallas as pl


def _dpn_body(x_ref, w0_ref, w1_ref, w2_ref, cb_ref, relw_ref, relb_ref,
              durw_ref, durb_ref, rel_out_ref, dur_out_ref):
    x = x_ref[0]
    a0 = jnp.dot(w0_ref[...], x, preferred_element_type=jnp.float32)
    a1 = jnp.dot(w1_ref[...], x, preferred_element_type=jnp.float32)
    a2 = jnp.dot(w2_ref[...], x, preferred_element_type=jnp.float32)
    zero_col = jnp.zeros((a0.shape[0], 1), dtype=a0.dtype)
    a0s = jnp.concatenate([zero_col, a0[:, :-1]], axis=1)
    a2s = jnp.concatenate([a2[:, 1:], zero_col], axis=1)
    t = jnp.maximum(a0s + a1 + a2s + cb_ref[...].T, 0.0)
    rel = jnp.dot(relw_ref[...], t, preferred_element_type=jnp.float32)
    dur = jnp.dot(durw_ref[...], t, preferred_element_type=jnp.float32)
    rel_out_ref[0] = rel + relb_ref[...].T
    dur_out_ref[0] = dur + durb_ref[...].T


def kernel(rel_feats, gt_rels, conv_w, conv_b, rel_w, rel_b, dur_w, dur_b):
    N, C, T = rel_feats.shape
    W = rel_w.shape[0]
    w0 = conv_w[:, :, 0]
    w1 = conv_w[:, :, 1]
    w2 = conv_w[:, :, 2]
    relw = rel_w[:, :, 0]
    durw = dur_w[:, :, 0]
    cb = conv_b[:, None].T
    relb = rel_b[:, None].T
    durb = dur_b[:, None].T

    full = lambda shp: pl.BlockSpec(shp, lambda n: (0,) * len(shp))
    grid = (N,)
    rel_out, dur_out = pl.pallas_call(
        _dpn_body,
        grid=grid,
        in_specs=[
            pl.BlockSpec((1, C, T), lambda n: (n, 0, 0)),
            full((C, C)), full((C, C)), full((C, C)), full((1, C)),
            full((W, C)), full((1, W)),
            full((2 * W, C)), full((1, 2 * W)),
        ],
        out_specs=[
            pl.BlockSpec((1, W, T), lambda n: (n, 0, 0)),
            pl.BlockSpec((1, 2 * W, T), lambda n: (n, 0, 0)),
        ],
        out_shape=[
            jax.ShapeDtypeStruct((N, W, T), jnp.float32),
            jax.ShapeDtypeStruct((N, 2 * W, T), jnp.float32),
        ],
    )(rel_feats, w0, w1, w2, cb, relw, relb, durw, durb)
    return (rel_out, dur_out)


# speedup vs baseline: 2.0188x; 2.0188x over previous
import jax
import jax.numpy as jnp
from jax.experimental import pallas as pl

_B = 8


def _dpn_body(x_ref, wstack_ref, cb_ref, hw_ref, hb_ref,
              rel_out_ref, dur_out_ref):
    wstack = wstack_ref[...]
    hw = hw_ref[...]
    cbt = cb_ref[...].T
    hbt = hb_ref[...].T
    C = wstack.shape[1]
    W = hw.shape[0] // 3
    for b in range(_B):
        x = x_ref[b].astype(jnp.bfloat16)
        a = jnp.dot(wstack, x, preferred_element_type=jnp.float32)
        a0, a1, a2 = a[:C], a[C:2 * C], a[2 * C:]
        zero_col = jnp.zeros((C, 1), dtype=a.dtype)
        a0s = jnp.concatenate([zero_col, a0[:, :-1]], axis=1)
        a2s = jnp.concatenate([a2[:, 1:], zero_col], axis=1)
        t = jnp.maximum(a0s + a1 + a2s + cbt, 0.0)
        h = jnp.dot(hw, t.astype(jnp.bfloat16),
                    preferred_element_type=jnp.float32) + hbt
        rel_out_ref[b] = h[:W]
        dur_out_ref[b] = h[W:]


def kernel(rel_feats, gt_rels, conv_w, conv_b, rel_w, rel_b, dur_w, dur_b):
    N, C, T = rel_feats.shape
    W = rel_w.shape[0]
    wstack = jnp.concatenate(
        [conv_w[:, :, 0], conv_w[:, :, 1], conv_w[:, :, 2]],
        axis=0).astype(jnp.bfloat16)
    hw = jnp.concatenate([rel_w[:, :, 0], dur_w[:, :, 0]],
                         axis=0).astype(jnp.bfloat16)
    cb = conv_b[None, :]
    hb = jnp.concatenate([rel_b, dur_b])[None, :]

    full = lambda shp: pl.BlockSpec(shp, lambda n: (0,) * len(shp))
    rel_out, dur_out = pl.pallas_call(
        _dpn_body,
        grid=(N // _B,),
        in_specs=[
            pl.BlockSpec((_B, C, T), lambda n: (n, 0, 0)),
            full((3 * C, C)), full((1, C)),
            full((3 * W, C)), full((1, 3 * W)),
        ],
        out_specs=[
            pl.BlockSpec((_B, W, T), lambda n: (n, 0, 0)),
            pl.BlockSpec((_B, 2 * W, T), lambda n: (n, 0, 0)),
        ],
        out_shape=[
            jax.ShapeDtypeStruct((N, W, T), jnp.float32),
            jax.ShapeDtypeStruct((N, 2 * W, T), jnp.float32),
        ],
    )(rel_feats, wstack, cb, hw, hb)
    return (rel_out, dur_out)


# speedup vs baseline: 3.4167x; 1.6925x over previous
import jax
import jax.numpy as jnp
from jax.experimental import pallas as pl

_B = 8


def _dpn_body(x_ref, wrow_ref, cb_ref, hw_ref, hb_ref,
              rel_out_ref, dur_out_ref):
    wrow = wrow_ref[...]
    hw = hw_ref[...]
    cbt = cb_ref[...].T
    hbt = hb_ref[...].T
    C = wrow.shape[0]
    W = hw.shape[0] // 3
    for b in range(_B):
        x = x_ref[b].astype(jnp.bfloat16)
        zero_col = jnp.zeros((C, 1), dtype=x.dtype)
        xr = jnp.concatenate([zero_col, x[:, :-1]], axis=1)
        xl = jnp.concatenate([x[:, 1:], zero_col], axis=1)
        xcat = jnp.concatenate([xr, x, xl], axis=0)
        conv = jnp.dot(wrow, xcat, preferred_element_type=jnp.float32)
        t = jnp.maximum(conv + cbt, 0.0)
        h = jnp.dot(hw, t.astype(jnp.bfloat16),
                    preferred_element_type=jnp.float32) + hbt
        rel_out_ref[b] = h[:W]
        dur_out_ref[b] = h[W:]


def kernel(rel_feats, gt_rels, conv_w, conv_b, rel_w, rel_b, dur_w, dur_b):
    N, C, T = rel_feats.shape
    W = rel_w.shape[0]
    wrow = jnp.concatenate(
        [conv_w[:, :, 0], conv_w[:, :, 1], conv_w[:, :, 2]],
        axis=1).astype(jnp.bfloat16)
    hw = jnp.concatenate([rel_w[:, :, 0], dur_w[:, :, 0]],
                         axis=0).astype(jnp.bfloat16)
    cb = conv_b[None, :]
    hb = jnp.concatenate([rel_b, dur_b])[None, :]

    full = lambda shp: pl.BlockSpec(shp, lambda n: (0,) * len(shp))
    rel_out, dur_out = pl.pallas_call(
        _dpn_body,
        grid=(N // _B,),
        in_specs=[
            pl.BlockSpec((_B, C, T), lambda n: (n, 0, 0)),
            full((C, 3 * C)), full((1, C)),
            full((3 * W, C)), full((1, 3 * W)),
        ],
        out_specs=[
            pl.BlockSpec((_B, W, T), lambda n: (n, 0, 0)),
            pl.BlockSpec((_B, 2 * W, T), lambda n: (n, 0, 0)),
        ],
        out_shape=[
            jax.ShapeDtypeStruct((N, W, T), jnp.float32),
            jax.ShapeDtypeStruct((N, 2 * W, T), jnp.float32),
        ],
    )(rel_feats, wrow, cb, hw, hb)
    return (rel_out, dur_out)


# speedup vs baseline: 4.7828x; 1.3998x over previous
import jax
import jax.numpy as jnp
from jax.experimental import pallas as pl

_B = 8


def _dpn_body(x_ref, wrow_ref, cb_ref, hw_ref, hb_ref,
              rel_out_ref, dur_out_ref):
    wrow = wrow_ref[...]
    hw = hw_ref[...]
    cbt = cb_ref[...].T
    hbt = hb_ref[...].T
    C = wrow.shape[0]
    W = hw.shape[0] // 3
    T = x_ref.shape[2]
    zero_col = jnp.zeros((C, 1), dtype=jnp.bfloat16)
    cols = []
    for b in range(_B):
        x = x_ref[b].astype(jnp.bfloat16)
        xr = jnp.concatenate([zero_col, x[:, :-1]], axis=1)
        xl = jnp.concatenate([x[:, 1:], zero_col], axis=1)
        cols.append(jnp.concatenate([xr, x, xl], axis=0))
    xcat = jnp.concatenate(cols, axis=1)
    conv = jnp.dot(wrow, xcat, preferred_element_type=jnp.float32)
    t = jnp.maximum(conv + cbt, 0.0)
    h = jnp.dot(hw, t.astype(jnp.bfloat16),
                preferred_element_type=jnp.float32) + hbt
    for b in range(_B):
        rel_out_ref[b] = h[:W, b * T:(b + 1) * T]
        dur_out_ref[b] = h[W:, b * T:(b + 1) * T]


def kernel(rel_feats, gt_rels, conv_w, conv_b, rel_w, rel_b, dur_w, dur_b):
    N, C, T = rel_feats.shape
    W = rel_w.shape[0]
    wrow = jnp.concatenate(
        [conv_w[:, :, 0], conv_w[:, :, 1], conv_w[:, :, 2]],
        axis=1).astype(jnp.bfloat16)
    hw = jnp.concatenate([rel_w[:, :, 0], dur_w[:, :, 0]],
                         axis=0).astype(jnp.bfloat16)
    cb = conv_b[None, :]
    hb = jnp.concatenate([rel_b, dur_b])[None, :]

    full = lambda shp: pl.BlockSpec(shp, lambda n: (0,) * len(shp))
    rel_out, dur_out = pl.pallas_call(
        _dpn_body,
        grid=(N // _B,),
        in_specs=[
            pl.BlockSpec((_B, C, T), lambda n: (n, 0, 0)),
            full((C, 3 * C)), full((1, C)),
            full((3 * W, C)), full((1, 3 * W)),
        ],
        out_specs=[
            pl.BlockSpec((_B, W, T), lambda n: (n, 0, 0)),
            pl.BlockSpec((_B, 2 * W, T), lambda n: (n, 0, 0)),
        ],
        out_shape=[
            jax.ShapeDtypeStruct((N, W, T), jnp.float32),
            jax.ShapeDtypeStruct((N, 2 * W, T), jnp.float32),
        ],
    )(rel_feats, wrow, cb, hw, hb)
    return (rel_out, dur_out)


# speedup vs baseline: 6.5832x; 1.3764x over previous
import jax
import jax.numpy as jnp
from jax.experimental import pallas as pl

_B = 16


def _dpn_body(x_ref, wrow_ref, cb_ref, hw_ref, hb_ref,
              rel_out_ref, dur_out_ref):
    wrow = wrow_ref[...]
    hw = hw_ref[...]
    cbt = cb_ref[...].T
    hbt = hb_ref[...].T
    C = wrow.shape[0]
    W = hw.shape[0] // 3
    T = x_ref.shape[2]
    zero_col = jnp.zeros((C, 1), dtype=jnp.bfloat16)
    cols = []
    for b in range(_B):
        x = x_ref[b].astype(jnp.bfloat16)
        xr = jnp.concatenate([zero_col, x[:, :-1]], axis=1)
        xl = jnp.concatenate([x[:, 1:], zero_col], axis=1)
        cols.append(jnp.concatenate([xr, x, xl], axis=0))
    xcat = jnp.concatenate(cols, axis=1)
    conv = jnp.dot(wrow, xcat, preferred_element_type=jnp.float32)
    t = jnp.maximum(conv + cbt, 0.0)
    h = jnp.dot(hw, t.astype(jnp.bfloat16),
                preferred_element_type=jnp.float32) + hbt
    for b in range(_B):
        rel_out_ref[b] = h[:W, b * T:(b + 1) * T]
        dur_out_ref[b] = h[W:, b * T:(b + 1) * T]


def kernel(rel_feats, gt_rels, conv_w, conv_b, rel_w, rel_b, dur_w, dur_b):
    N, C, T = rel_feats.shape
    W = rel_w.shape[0]
    wrow = jnp.concatenate(
        [conv_w[:, :, 0], conv_w[:, :, 1], conv_w[:, :, 2]],
        axis=1).astype(jnp.bfloat16)
    hw = jnp.concatenate([rel_w[:, :, 0], dur_w[:, :, 0]],
                         axis=0).astype(jnp.bfloat16)
    cb = conv_b[None, :]
    hb = jnp.concatenate([rel_b, dur_b])[None, :]

    full = lambda shp: pl.BlockSpec(shp, lambda n: (0,) * len(shp))
    rel_out, dur_out = pl.pallas_call(
        _dpn_body,
        grid=(N // _B,),
        in_specs=[
            pl.BlockSpec((_B, C, T), lambda n: (n, 0, 0)),
            full((C, 3 * C)), full((1, C)),
            full((3 * W, C)), full((1, 3 * W)),
        ],
        out_specs=[
            pl.BlockSpec((_B, W, T), lambda n: (n, 0, 0)),
            pl.BlockSpec((_B, 2 * W, T), lambda n: (n, 0, 0)),
        ],
        out_shape=[
            jax.ShapeDtypeStruct((N, W, T), jnp.float32),
            jax.ShapeDtypeStruct((N, 2 * W, T), jnp.float32),
        ],
    )(rel_feats, wrow, cb, hw, hb)
    return (rel_out, dur_out)


# speedup vs baseline: 10.0385x; 1.5249x over previous
import jax
import jax.numpy as jnp
from jax.experimental import pallas as pl

_B = 32


def _dpn_body(x_ref, wrow_ref, cb_ref, hw_ref, hb_ref,
              rel_out_ref, dur_out_ref):
    wrow = wrow_ref[...]
    hw = hw_ref[...]
    cbt = cb_ref[...].T
    hbt = hb_ref[...].T
    C = wrow.shape[0]
    W = hw.shape[0] // 3
    T = x_ref.shape[2]
    zero_col = jnp.zeros((C, 1), dtype=jnp.bfloat16)
    cols = []
    for b in range(_B):
        x = x_ref[b].astype(jnp.bfloat16)
        xr = jnp.concatenate([zero_col, x[:, :-1]], axis=1)
        xl = jnp.concatenate([x[:, 1:], zero_col], axis=1)
        cols.append(jnp.concatenate([xr, x, xl], axis=0))
    xcat = jnp.concatenate(cols, axis=1)
    conv = jnp.dot(wrow, xcat, preferred_element_type=jnp.float32)
    t = jnp.maximum(conv.astype(jnp.bfloat16) + cbt.astype(jnp.bfloat16),
                    jnp.bfloat16(0.0))
    h = jnp.dot(hw, t, preferred_element_type=jnp.float32) + hbt
    for b in range(_B):
        rel_out_ref[b] = h[:W, b * T:(b + 1) * T]
        dur_out_ref[b] = h[W:, b * T:(b + 1) * T]


def kernel(rel_feats, gt_rels, conv_w, conv_b, rel_w, rel_b, dur_w, dur_b):
    N, C, T = rel_feats.shape
    W = rel_w.shape[0]
    wrow = jnp.concatenate(
        [conv_w[:, :, 0], conv_w[:, :, 1], conv_w[:, :, 2]],
        axis=1).astype(jnp.bfloat16)
    hw = jnp.concatenate([rel_w[:, :, 0], dur_w[:, :, 0]],
                         axis=0).astype(jnp.bfloat16)
    cb = conv_b[None, :]
    hb = jnp.concatenate([rel_b, dur_b])[None, :]

    full = lambda shp: pl.BlockSpec(shp, lambda n: (0,) * len(shp))
    rel_out, dur_out = pl.pallas_call(
        _dpn_body,
        grid=(N // _B,),
        in_specs=[
            pl.BlockSpec((_B, C, T), lambda n: (n, 0, 0)),
            full((C, 3 * C)), full((1, C)),
            full((3 * W, C)), full((1, 3 * W)),
        ],
        out_specs=[
            pl.BlockSpec((_B, W, T), lambda n: (n, 0, 0)),
            pl.BlockSpec((_B, 2 * W, T), lambda n: (n, 0, 0)),
        ],
        out_shape=[
            jax.ShapeDtypeStruct((N, W, T), jnp.float32),
            jax.ShapeDtypeStruct((N, 2 * W, T), jnp.float32),
        ],
    )(rel_feats, wrow, cb, hw, hb)
    return (rel_out, dur_out)
